# baseline (speedup 1.0000x reference)
import functools

import jax
import jax.numpy as jnp
from jax.experimental import pallas as pl

NUM_HIDDENS = 128
NUM_RES_HIDDENS = 32
EMB_DIM = 64
NUM_EMB = 512
COMMIT = 0.25

_F32 = jnp.float32


_CONV_MM = "x1"


def _mm(x, w, mode=None):
    mode = mode or _CONV_MM
    if mode == "hi":
        return jnp.dot(x, w, preferred_element_type=_F32,
                       precision=jax.lax.Precision.HIGHEST)
    if mode == "x1":
        return jnp.dot(x, w, preferred_element_type=_F32, precision=None)
    xh = x.astype(jnp.bfloat16).astype(_F32)
    xl = x - xh
    wh = w.astype(jnp.bfloat16).astype(_F32)
    wl = w - wh
    d = lambda a, b: jnp.dot(a, b, preferred_element_type=_F32,
                             precision=None)
    if mode == "xw2":
        return d(xl, wh) + d(xh, wh)
    if mode == "wx2":
        return d(xh, wl) + d(xh, wh)
    return d(xl, wh) + (d(xh, wl) + d(xh, wh))


def _mm_t(w, x, mode=None):
    mode = mode or _CONV_MM
    dims = (((0,), (1,)), ((), ()))
    if mode == "hi":
        return jax.lax.dot_general(w, x, dims, preferred_element_type=_F32,
                                   precision=jax.lax.Precision.HIGHEST)
    if mode == "x1":
        return jax.lax.dot_general(w, x, dims, preferred_element_type=_F32,
                                   precision=None)
    xh = x.astype(jnp.bfloat16).astype(_F32)
    xl = x - xh
    wh = w.astype(jnp.bfloat16).astype(_F32)
    wl = w - wh
    d = lambda a, b: jax.lax.dot_general(a, b, dims,
                                         preferred_element_type=_F32,
                                         precision=None)
    return d(wh, xl) + (d(wl, xh) + d(wh, xh))


def _pad_hw(x, p=1):
    return jnp.pad(x, ((0, 0), (p, p), (p, p), (0, 0)))


def _s2d(x):
    n, h, w, c = x.shape
    x = x.reshape(n, h // 2, 2, w // 2, 2, c)
    x = x.transpose(0, 1, 3, 2, 4, 5)
    return x.reshape(n, h // 2, w // 2, 4 * c)


def _im2col_s2(xh, k=4):
    xp = _pad_hw(xh, 1)
    ho = (xh.shape[1] + 2 - k) // 2 + 1
    cols = []
    for ky in range(k):
        for kx in range(k):
            cols.append(jax.lax.slice(
                xp, (0, ky, kx, 0),
                (xp.shape[0], ky + 2 * (ho - 1) + 1, kx + 2 * (ho - 1) + 1,
                 xp.shape[3]), (1, 2, 2, 1)))
    return jnp.concatenate(cols, axis=-1)


def _w_flat_s2(w):
    o, i, kh, kw = w.shape
    return w.transpose(2, 3, 1, 0).reshape(1, kh * kw * i, o)


def _w_taps_3x3(w):
    o, i, kh, kw = w.shape
    return w.transpose(2, 3, 1, 0).reshape(kh * kw, i, o)


def _w_taps_s2(w):
    o, i, _, _ = w.shape
    wt = w.transpose(2, 3, 1, 0)
    wt = wt.reshape(2, 2, 2, 2, i, o)
    wt = wt.transpose(0, 2, 1, 3, 4, 5)
    return wt.reshape(4, 4 * i, o)


def _conv_taps_kernel(x_ref, w_ref, b_ref, o_ref, *, offsets, Ho, Wo,
                      relu_in, relu_out, rchunk, mode=None):
    for r0 in range(0, Ho, rchunk):
        acc = None
        for t, (dy, dx) in enumerate(offsets):
            xs = x_ref[0, r0 + dy:r0 + dy + rchunk, dx:dx + Wo, :]
            if relu_in:
                xs = jnp.maximum(xs, 0.0)
            xs = xs.reshape(rchunk * Wo, xs.shape[-1])
            p = _mm(xs, w_ref[t], mode)
            acc = p if acc is None else acc + p
        acc = acc + b_ref[0][None, :]
        if relu_out:
            acc = jnp.maximum(acc, 0.0)
        o_ref[0, r0:r0 + rchunk] = acc.reshape(rchunk, Wo, acc.shape[-1])


def _conv_taps(xpad, w_taps, b, offsets, Ho, Wo, relu_in=False, relu_out=False, mode=None):
    n, hp, wp, cin = xpad.shape
    t, _, cout = w_taps.shape
    return pl.pallas_call(
        functools.partial(_conv_taps_kernel, offsets=offsets, Ho=Ho, Wo=Wo,
                          relu_in=relu_in, relu_out=relu_out,
                          rchunk=28 if Ho > 56 else Ho, mode=mode),
        grid=(n,),
        in_specs=[
            pl.BlockSpec((1, hp, wp, cin), lambda i: (i, 0, 0, 0)),
            pl.BlockSpec((t, w_taps.shape[1], cout), lambda i: (0, 0, 0)),
            pl.BlockSpec((1, cout), lambda i: (0, 0)),
        ],
        out_specs=pl.BlockSpec((1, Ho, Wo, cout), lambda i: (i, 0, 0, 0)),
        out_shape=jax.ShapeDtypeStruct((n, Ho, Wo, cout), _F32),
    )(xpad, w_taps, b.reshape(1, cout))


_OFFS_2X2 = tuple((dy, dx) for dy in range(2) for dx in range(2))
_OFFS_3X3 = tuple((dy, dx) for dy in range(3) for dx in range(3))


def _res_block_kernel(x_ref, w1_ref, w2_ref, o_ref, *, H, W, final_relu, mode=None):
    acc = None
    for t, (dy, dx) in enumerate(_OFFS_3X3):
        xs = jnp.maximum(x_ref[0, dy:dy + H, dx:dx + W, :], 0.0)
        xs = xs.reshape(H * W, xs.shape[-1])
        p = _mm(xs, w1_ref[t], mode)
        acc = p if acc is None else acc + p
    h = jnp.maximum(acc, 0.0)
    h2 = _mm(h, w2_ref[...], mode)
    xc = x_ref[0, 1:1 + H, 1:1 + W, :].reshape(H * W, h2.shape[-1])
    out = xc + h2
    if final_relu:
        out = jnp.maximum(out, 0.0)
    o_ref[0] = out.reshape(H, W, out.shape[-1])


def _res_block(xpad, w1, w2, final_relu=False, mode=None):
    n, hp, wp, c = xpad.shape
    H, W = hp - 2, wp - 2
    w1t = _w_taps_3x3(w1)
    w2t = w2[:, :, 0, 0].T
    return pl.pallas_call(
        functools.partial(_res_block_kernel, H=H, W=W, final_relu=final_relu, mode=mode),
        grid=(n,),
        in_specs=[
            pl.BlockSpec((1, hp, wp, c), lambda i: (i, 0, 0, 0)),
            pl.BlockSpec(w1t.shape, lambda i: (0, 0, 0)),
            pl.BlockSpec(w2t.shape, lambda i: (0, 0)),
        ],
        out_specs=pl.BlockSpec((1, H, W, c), lambda i: (i, 0, 0, 0)),
        out_shape=jax.ShapeDtypeStruct((n, H, W, c), _F32),
    )(xpad, w1t, w2t)


def _res_block_pv_kernel(x_ref, w1_ref, w2_ref, pvw_ref, pvb_ref, o_ref, *,
                         H, W, mode=None):
    acc = None
    for t, (dy, dx) in enumerate(_OFFS_3X3):
        xs = jnp.maximum(x_ref[0, dy:dy + H, dx:dx + W, :], 0.0)
        xs = xs.reshape(H * W, xs.shape[-1])
        p = _mm(xs, w1_ref[t], mode)
        acc = p if acc is None else acc + p
    h = jnp.maximum(acc, 0.0)
    h2 = _mm(h, w2_ref[...], mode)
    xc = x_ref[0, 1:1 + H, 1:1 + W, :].reshape(H * W, h2.shape[-1])
    out = jnp.maximum(xc + h2, 0.0)
    z = _mm(out, pvw_ref[...], mode) + pvb_ref[0][None, :]
    o_ref[0] = z.reshape(H, W, z.shape[-1])


def _res_block_pv(xpad, w1, w2, pv_w, pv_b, mode=None):
    n, hp, wp, c = xpad.shape
    H, W = hp - 2, wp - 2
    w1t = _w_taps_3x3(w1)
    w2t = w2[:, :, 0, 0].T
    pvt = pv_w[:, :, 0, 0].T
    cout = pvt.shape[1]
    return pl.pallas_call(
        functools.partial(_res_block_pv_kernel, H=H, W=W, mode=mode),
        grid=(n,),
        in_specs=[
            pl.BlockSpec((1, hp, wp, c), lambda i: (i, 0, 0, 0)),
            pl.BlockSpec(w1t.shape, lambda i: (0, 0, 0)),
            pl.BlockSpec(w2t.shape, lambda i: (0, 0)),
            pl.BlockSpec(pvt.shape, lambda i: (0, 0)),
            pl.BlockSpec((1, cout), lambda i: (0, 0)),
        ],
        out_specs=pl.BlockSpec((1, H, W, cout), lambda i: (i, 0, 0, 0)),
        out_shape=jax.ShapeDtypeStruct((n, H, W, cout), _F32),
    )(xpad, w1t, w2t, pvt, pv_b.reshape(1, cout))


_PH_OFF = ((0, 1), (1, 2))
_PH_K = ((3, 1), (2, 0))


def _convt_phase_kernel(x_ref, w_ref, b_ref, o00, o01, o10, o11, *, Ho, Wo,
                        relu_out, rchunk):
    outs = ((o00, o01), (o10, o11))
    for a in range(2):
        for b in range(2):
            for r0 in range(0, Ho, rchunk):
                acc = None
                for ti in range(2):
                    dy = _PH_OFF[a][ti]
                    for tj in range(2):
                        dx = _PH_OFF[b][tj]
                        xs = x_ref[0, r0 + dy:r0 + dy + rchunk,
                                   dx:dx + Wo, :]
                        xs = xs.reshape(rchunk * Wo, xs.shape[-1])
                        p = _mm(xs, w_ref[a, b, ti, tj])
                        acc = p if acc is None else acc + p
                acc = acc + b_ref[0][None, :]
                if relu_out:
                    acc = jnp.maximum(acc, 0.0)
                outs[a][b][0, r0:r0 + rchunk] = acc.reshape(
                    rchunk, Wo, acc.shape[-1])


def _phase_weights(w):
    wt = w.transpose(2, 3, 0, 1)
    return jnp.stack([
        jnp.stack([
            jnp.stack([
                jnp.stack([wt[_PH_K[a][ti], _PH_K[b][tj]] for tj in range(2)])
                for ti in range(2)])
            for b in range(2)])
        for a in range(2)])


def _conv_transpose(x, w, bias, relu_out):
    n, H, W, cin = x.shape
    cout = w.shape[1]
    xpad = _pad_hw(x, 1)
    wp = _phase_weights(w)
    outs = pl.pallas_call(
        functools.partial(_convt_phase_kernel, Ho=H, Wo=W, relu_out=relu_out,
                          rchunk=28 if H > 56 else H),
        grid=(n,),
        in_specs=[
            pl.BlockSpec((1, H + 2, W + 2, cin), lambda i: (i, 0, 0, 0)),
            pl.BlockSpec(wp.shape, lambda i: (0, 0, 0, 0, 0, 0)),
            pl.BlockSpec((1, cout), lambda i: (0, 0)),
        ],
        out_specs=[pl.BlockSpec((1, H, W, cout), lambda i: (i, 0, 0, 0))] * 4,
        out_shape=[jax.ShapeDtypeStruct((n, H, W, cout), _F32)] * 4,
    )(xpad, wp, bias.reshape(1, cout))
    s = jnp.stack(outs).reshape(2, 2, n, H, W, cout)
    s = s.transpose(2, 3, 0, 4, 1, 5)
    return s.reshape(n, 2 * H, 2 * W, cout)


def _convt_nchw_kernel(x_ref, w_ref, b_ref, o_ref, *, Ho, Wo, rchunk):
    for a in range(2):
        for b in range(2):
            for r0 in range(0, Ho, rchunk):
                acc = None
                for ti in range(2):
                    dy = _PH_OFF[a][ti]
                    for tj in range(2):
                        dx = _PH_OFF[b][tj]
                        xs = x_ref[0, r0 + dy:r0 + dy + rchunk,
                                   dx:dx + Wo, :]
                        xs = xs.reshape(rchunk * Wo, xs.shape[-1])
                        p = _mm_t(w_ref[a, b, ti, tj], xs)
                        acc = p if acc is None else acc + p
                acc = acc + b_ref[...]
                o_ref[0, a, b, :, r0:r0 + rchunk] = acc.reshape(
                    acc.shape[0], rchunk, Wo)


def _conv_transpose_to_nchw(x, w, bias):
    n, H, W, cin = x.shape
    cout = w.shape[1]
    xpad = _pad_hw(x, 1)
    wp = _phase_weights(w)
    out = pl.pallas_call(
        functools.partial(_convt_nchw_kernel, Ho=H, Wo=W,
                          rchunk=28 if H > 56 else H),
        grid=(n,),
        in_specs=[
            pl.BlockSpec((1, H + 2, W + 2, cin), lambda i: (i, 0, 0, 0)),
            pl.BlockSpec(wp.shape, lambda i: (0, 0, 0, 0, 0, 0)),
            pl.BlockSpec((cout, 1), lambda i: (0, 0)),
        ],
        out_specs=pl.BlockSpec((1, 2, 2, cout, H, W),
                               lambda i: (i, 0, 0, 0, 0, 0)),
        out_shape=jax.ShapeDtypeStruct((n, 2, 2, cout, H, W), _F32),
    )(xpad, wp, bias.reshape(cout, 1))
    s = out.transpose(0, 3, 4, 1, 5, 2)
    return s.reshape(n, cout, 2 * H, 2 * W)


def _vq_kernel(z_ref, cb_ref, q_ref, cnt_ref, loss_ref, perp_ref, *,
               steps, total_vecs, total_elems):
    i = pl.program_id(0)
    z = z_ref[...]
    cb = cb_ref[...]
    z2 = jnp.sum(z * z, axis=1, keepdims=True)
    c2r = jax.lax.dot_general(
        jnp.ones((1, cb.shape[1]), _F32), cb * cb, (((1,), (1,)), ((), ())),
        preferred_element_type=_F32,
        precision=jax.lax.Precision.HIGHEST)
    zc = jax.lax.dot_general(z, cb, (((1,), (1,)), ((), ())),
                             preferred_element_type=_F32, precision=None)
    d = (z2 + c2r) - 2.0 * zc
    m = jnp.min(d, axis=1, keepdims=True)
    iota = jax.lax.broadcasted_iota(jnp.int32, d.shape, 1)
    idx = jnp.min(jnp.where(d == m, iota, NUM_EMB), axis=1)
    oh = (iota == idx[:, None]).astype(_F32)
    q = jnp.dot(oh, cb, preferred_element_type=_F32, precision=None)
    q_ref[...] = q

    cnt_p = jnp.sum(oh, axis=0)[None, :]
    loss_p = jnp.sum((q - z) ** 2).reshape(1, 1)

    @pl.when(i == 0)
    def _init():
        cnt_ref[...] = cnt_p
        loss_ref[...] = loss_p

    @pl.when(i > 0)
    def _acc():
        cnt_ref[...] = cnt_ref[...] + cnt_p
        loss_ref[...] = loss_ref[...] + loss_p

    @pl.when(i == steps - 1)
    def _finish():
        avg = cnt_ref[...] / total_vecs
        perp_ref[...] = jnp.exp(
            -jnp.sum(avg * jnp.log(avg + 1e-10))).reshape(1, 1)
        loss_ref[...] = loss_ref[...] * (COMMIT / total_elems)


def _vq(z_flat, codebook):
    M, D = z_flat.shape
    TM = 512
    steps = M // TM
    q, cnt, loss, perp = pl.pallas_call(
        functools.partial(_vq_kernel, steps=steps, total_vecs=float(M),
                          total_elems=float(M * D)),
        grid=(steps,),
        in_specs=[
            pl.BlockSpec((TM, D), lambda i: (i, 0)),
            pl.BlockSpec((NUM_EMB, D), lambda i: (0, 0)),
        ],
        out_specs=[
            pl.BlockSpec((TM, D), lambda i: (i, 0)),
            pl.BlockSpec((1, NUM_EMB), lambda i: (0, 0)),
            pl.BlockSpec((1, 1), lambda i: (0, 0)),
            pl.BlockSpec((1, 1), lambda i: (0, 0)),
        ],
        out_shape=[
            jax.ShapeDtypeStruct((M, D), _F32),
            jax.ShapeDtypeStruct((1, NUM_EMB), _F32),
            jax.ShapeDtypeStruct((1, 1), _F32),
            jax.ShapeDtypeStruct((1, 1), _F32),
        ],
    )(z_flat, codebook)
    return q, loss[0, 0], perp[0, 0]


def kernel(x, e1_w, e1_b, e2_w, e2_b, e3_w, e3_b, er1_w1, er1_w2, er2_w1,
           er2_w2, pv_w, pv_b, codebook, d1_w, d1_b, dr1_w1, dr1_w2, dr2_w1,
           dr2_w2, dt1_w, dt1_b, dt2_w, dt2_b):
    n = x.shape[0]
    xh = x.transpose(0, 2, 3, 1)

    h = _conv_taps(_im2col_s2(xh), _w_flat_s2(e1_w), e1_b, ((0, 0),),
                   112, 112, relu_out=True)
    h = _conv_taps(_im2col_s2(h), _w_flat_s2(e2_w), e2_b, ((0, 0),),
                   56, 56, relu_out=True)
    h = _conv_taps(_pad_hw(h), _w_taps_3x3(e3_w), e3_b, _OFFS_3X3, 56, 56)
    h = _res_block(_pad_hw(h), er1_w1, er1_w2)
    z = _res_block_pv(_pad_hw(h), er2_w1, er2_w2, pv_w, pv_b)

    q, loss, perp = _vq(z.reshape(-1, EMB_DIM), codebook)
    q = q.reshape(n, 56, 56, EMB_DIM)

    h = _conv_taps(_pad_hw(q), _w_taps_3x3(d1_w), d1_b, _OFFS_3X3, 56, 56)
    h = _res_block(_pad_hw(h), dr1_w1, dr1_w2)
    h = _res_block(_pad_hw(h), dr2_w1, dr2_w2, final_relu=True)
    h = _conv_transpose(h, dt1_w, dt1_b, relu_out=True)
    xr = _conv_transpose_to_nchw(h, dt2_w, dt2_b)

    return loss, xr, perp
